# baseline (speedup 1.0000x reference)
import jax
import jax.numpy as jnp
from jax import lax
from jax.experimental import pallas as pl
from jax.experimental.pallas import tpu as pltpu
from jax.experimental.pallas import tpu_sc as plsc

VOCAB = 1000000
EMBED_DIM = 64
BATCH = 4096
HIST = 50

NC, NS = 2, 16
NW = NC * NS
B_CH = 8
CHUNK = B_CH * HIST
N_IDX = BATCH * HIST
CPW = N_IDX // (NW * CHUNK)
BPW = BATCH // NW
HP = 56
DP = 128

_mesh = plsc.VectorSubcoreMesh(core_axis_name="c", subcore_axis_name="s",
                               num_cores=NC, num_subcores=NS)


NBUF = 4


def _body(idx_hbm, tbl_hbm, out_hbm, idx_v, rows0, rows1, rows2, rows3,
          gsem0, gsem1, gsem2, gsem3, osem0, osem1, osem2, osem3):
    wid = lax.axis_index("s") * NC + lax.axis_index("c")
    bbase = wid * BPW
    pltpu.sync_copy(idx_hbm.at[wid], idx_v)

    rows = (rows0, rows1, rows2, rows3)
    gsem = (gsem0, gsem1, gsem2, gsem3)
    osem = (osem0, osem1, osem2, osem3)

    def gather(j, b):
        return pltpu.async_copy(tbl_hbm.at[idx_v.at[j]], rows[b], gsem[b])

    def outcopy(j, b):
        descs = []
        for k in range(B_CH):
            descs.append(pltpu.async_copy(
                rows[b].at[pl.ds(k * HIST, HIST)],
                out_hbm.at[bbase + j * B_CH + k, pl.ds(0, HIST),
                           pl.ds(0, EMBED_DIM)],
                osem[b]))
        return descs

    def wait_all(descs):
        for d in descs:
            d.wait()

    g = [None] * NBUF
    o = [None] * NBUF
    for p in range(NBUF - 1):
        g[p] = gather(p, p)
    for j in range(CPW):
        b = j % NBUF
        fb = (j + NBUF - 1) % NBUF
        if j + NBUF - 1 < CPW:
            if o[fb] is not None:
                wait_all(o[fb])
            g[fb] = gather(j + NBUF - 1, fb)
        g[b].wait()
        o[b] = outcopy(j, b)
    for b in range(NBUF):
        if o[b] is not None:
            wait_all(o[b])


_gather = pl.kernel(
    _body,
    out_type=jax.ShapeDtypeStruct((BATCH, HP, DP), jnp.float32),
    mesh=_mesh,
    scratch_types=[
        pltpu.VMEM((CPW, CHUNK), jnp.int32),
        pltpu.VMEM((CHUNK, EMBED_DIM), jnp.float32),
        pltpu.VMEM((CHUNK, EMBED_DIM), jnp.float32),
        pltpu.VMEM((CHUNK, EMBED_DIM), jnp.float32),
        pltpu.VMEM((CHUNK, EMBED_DIM), jnp.float32),
        pltpu.SemaphoreType.DMA,
        pltpu.SemaphoreType.DMA,
        pltpu.SemaphoreType.DMA,
        pltpu.SemaphoreType.DMA,
        pltpu.SemaphoreType.DMA,
        pltpu.SemaphoreType.DMA,
        pltpu.SemaphoreType.DMA,
        pltpu.SemaphoreType.DMA,
    ],
    compiler_params=pltpu.CompilerParams(use_tc_tiling_on_sc=False),
)


def kernel(input, weight):
    idx = input.reshape(NW, CPW, CHUNK).astype(jnp.int32)
    padded = _gather(idx, weight)
    return padded[:, :HIST, :EMBED_DIM]
